# baseline (speedup 1.0000x reference)
import jax
import jax.numpy as jnp
from jax import lax
from jax.experimental import pallas as pl
from jax.experimental.pallas import tpu as pltpu
from jax.experimental.pallas import tpu_sc as plsc

N_DIM = 32
N_BIN = 32
N_NODE = N_BIN + 1
N_FP = 2 * N_BIN + 1
EPS_MIN = 1e-06
EPS_MAX = 1.0 - EPS_MIN
L = 16
NC = 2
NS = 16
NW = NC * NS
NPAR = N_DIM // L


def _spline_body(x_hbm, xnt_hbm, fpt_hbm, f_hbm, t_hbm, xin_v, xnt_v, fpt_v, f_v, t_v):
    rows = x_hbm.shape[0] // NW
    crows = xin_v.shape[0]
    wid = lax.axis_index("c") * NS + lax.axis_index("s")
    base = wid * rows

    pltpu.sync_copy(xnt_hbm, xnt_v)
    pltpu.sync_copy(fpt_hbm, fpt_v)

    iota = lax.iota(jnp.int32, L)
    niotas = [iota + g * N_NODE * L for g in range(NPAR)]
    fiotas = [iota + g * N_FP * L for g in range(NPAR)]
    ntree = []
    for g in range(NPAR):
        gb = g * N_NODE * L
        ntree.append({j: xnt_v[pl.ds(gb + j * L, L)] for j in (16, 8, 24, 4, 12, 20, 28)})

    def do_group(row, g):
        niota = niotas[g]
        fiota = fiotas[g]
        tr = ntree[g]
        x = xin_v[row, pl.ds(g * L, L)]
        xc = jnp.minimum(jnp.maximum(x, EPS_MIN), EPS_MAX)
        c1 = tr[16] < xc
        spos = jnp.where(c1, 16 * L, 0)
        nv = jnp.where(c1, tr[24], tr[8])
        c2 = nv < xc
        spos = jnp.where(c2, spos + 8 * L, spos)
        nv = jnp.where(c1, jnp.where(c2, tr[28], tr[20]), jnp.where(c2, tr[12], tr[4]))
        c3 = nv < xc
        spos = jnp.where(c3, spos + 4 * L, spos)
        for dl in (2 * L, L):
            probe = spos + dl
            nv = plsc.load_gather(xnt_v, [probe + niota])
            spos = jnp.where(nv < xc, probe, spos)
        n0i = spos + niota
        xn0 = plsc.load_gather(xnt_v, [n0i])
        xn1 = plsc.load_gather(xnt_v, [n0i + L])
        w0i = spos + fiota
        w0 = plsc.load_gather(fpt_v, [w0i])
        w2 = plsc.load_gather(fpt_v, [w0i + L])
        w1 = plsc.load_gather(fpt_v, [w0i + N_NODE * L])
        d = xn1 - xn0
        r = 1.0 / d
        r = r * (2.0 - d * r)
        t = (xc - xn0) * r
        w14 = 4.0 * w1
        a = w14 - 3.0 * w0 - w2
        s2 = w0 + w2
        b = (s2 + s2) - w14
        f = w0 + t * (a + t * b)
        t_v[row, pl.ds(g * L, L)] = t
        f_v[row, pl.ds(g * L, L)] = f

    for c in range(rows // crows):
        cbase = base + c * crows
        pltpu.sync_copy(x_hbm.at[pl.ds(cbase, crows), :], xin_v)

        @plsc.parallel_loop(0, crows, step=2)
        def _row_loop(row):
            for rr in range(2):
                for g in range(NPAR):
                    do_group(row + rr, g)

        pltpu.sync_copy(f_v, f_hbm.at[pl.ds(cbase, crows), :])
        pltpu.sync_copy(t_v, t_hbm.at[pl.ds(cbase, crows), :])


def kernel(x_in, x_node, f_params):
    batch, n_dim = x_in.shape
    rows = batch // NW
    xnt = x_node.reshape(NPAR, L, N_NODE).transpose(0, 2, 1).reshape(-1)
    fpt = f_params.reshape(NPAR, L, N_FP).transpose(0, 2, 1).reshape(-1)
    mesh = plsc.VectorSubcoreMesh(
        core_axis_name="c", subcore_axis_name="s", num_cores=NC, num_subcores=NS
    )
    f_out, t_out = pl.kernel(
        _spline_body,
        out_type=(
            jax.ShapeDtypeStruct((batch, n_dim), jnp.float32),
            jax.ShapeDtypeStruct((batch, n_dim), jnp.float32),
        ),
        mesh=mesh,
        compiler_params=pltpu.CompilerParams(
            needs_layout_passes=False, use_tc_tiling_on_sc=True
        ),
        scratch_types=[
            pltpu.VMEM((rows // 4, n_dim), jnp.float32),
            pltpu.VMEM((xnt.size,), jnp.float32),
            pltpu.VMEM((fpt.size,), jnp.float32),
            pltpu.VMEM((rows // 4, n_dim), jnp.float32),
            pltpu.VMEM((rows // 4, n_dim), jnp.float32),
        ],
    )(x_in, xnt, fpt)
    return f_out, t_out


# speedup vs baseline: 1.0992x; 1.0992x over previous
import jax
import jax.numpy as jnp
from jax import lax
from jax.experimental import pallas as pl
from jax.experimental.pallas import tpu as pltpu
from jax.experimental.pallas import tpu_sc as plsc

N_DIM = 32
N_BIN = 32
N_NODE = N_BIN + 1
N_FP = 2 * N_BIN + 1
EPS_MIN = 1e-06
EPS_MAX = 1.0 - EPS_MIN
L = 16
NC = 2
NS = 16
NW = NC * NS
NPAR = N_DIM // L


def _spline_body(
    x_hbm, xnt_hbm, fpt_hbm, f_hbm, t_hbm, xin_v, xnt_v, fpt_v, f_v, t_v, sem
):
    rows = x_hbm.shape[0] // NW
    wid = lax.axis_index("c") * NS + lax.axis_index("s")
    base = wid * rows

    d1 = pltpu.async_copy(x_hbm.at[pl.ds(base, rows), pl.ds(0, N_DIM)], xin_v, sem)
    d2 = pltpu.async_copy(xnt_hbm, xnt_v, sem)
    d3 = pltpu.async_copy(fpt_hbm, fpt_v, sem)
    d1.wait()
    d2.wait()
    d3.wait()

    iota = lax.iota(jnp.int32, L)
    niotas = [iota + g * N_NODE * L for g in range(NPAR)]
    fiotas = [iota + g * N_FP * L for g in range(NPAR)]
    ntree = []
    for g in range(NPAR):
        gb = g * N_NODE * L
        ntree.append({j: xnt_v[pl.ds(gb + j * L, L)] for j in (16, 8, 24, 4, 12, 20, 28)})

    def do_group(row, g):
        niota = niotas[g]
        fiota = fiotas[g]
        tr = ntree[g]
        x = xin_v[row, pl.ds(g * L, L)]
        xc = jnp.minimum(jnp.maximum(x, EPS_MIN), EPS_MAX)
        c1 = tr[16] < xc
        spos = jnp.where(c1, 16 * L, 0)
        nv = jnp.where(c1, tr[24], tr[8])
        c2 = nv < xc
        spos = jnp.where(c2, spos + 8 * L, spos)
        nv = jnp.where(c1, jnp.where(c2, tr[28], tr[20]), jnp.where(c2, tr[12], tr[4]))
        c3 = nv < xc
        spos = jnp.where(c3, spos + 4 * L, spos)
        for dl in (2 * L, L):
            probe = spos + dl
            nv = plsc.load_gather(xnt_v, [probe + niota])
            spos = jnp.where(nv < xc, probe, spos)
        n0i = spos + niota
        xn0 = plsc.load_gather(xnt_v, [n0i])
        xn1 = plsc.load_gather(xnt_v, [n0i + L])
        w0i = spos + fiota
        w0 = plsc.load_gather(fpt_v, [w0i])
        w2 = plsc.load_gather(fpt_v, [w0i + L])
        w1 = plsc.load_gather(fpt_v, [w0i + N_NODE * L])
        d = xn1 - xn0
        r = 1.0 / d
        r = r * (2.0 - d * r)
        t = (xc - xn0) * r
        w14 = 4.0 * w1
        a = w14 - 3.0 * w0 - w2
        s2 = w0 + w2
        b = (s2 + s2) - w14
        f = w0 + t * (a + t * b)
        t_v[row, pl.ds(g * L, L)] = t
        f_v[row, pl.ds(g * L, L)] = f

    @plsc.parallel_loop(0, rows, step=2)
    def _row_loop(row):
        for rr in range(2):
            for g in range(NPAR):
                do_group(row + rr, g)

    o1 = pltpu.async_copy(f_v, f_hbm.at[pl.ds(base, rows), pl.ds(0, N_DIM)], sem)
    o2 = pltpu.async_copy(t_v, t_hbm.at[pl.ds(base, rows), pl.ds(0, N_DIM)], sem)
    o1.wait()
    o2.wait()


def kernel(x_in, x_node, f_params):
    batch, n_dim = x_in.shape
    rows = batch // NW
    xnt = x_node.reshape(NPAR, L, N_NODE).transpose(0, 2, 1).reshape(-1)
    fpt = f_params.reshape(NPAR, L, N_FP).transpose(0, 2, 1).reshape(-1)
    mesh = plsc.VectorSubcoreMesh(
        core_axis_name="c", subcore_axis_name="s", num_cores=NC, num_subcores=NS
    )
    x128 = jnp.pad(x_in, ((0, 0), (0, 128 - n_dim)))
    f128, t128 = pl.kernel(
        _spline_body,
        out_type=(
            jax.ShapeDtypeStruct((batch, 128), jnp.float32),
            jax.ShapeDtypeStruct((batch, 128), jnp.float32),
        ),
        mesh=mesh,
        compiler_params=pltpu.CompilerParams(
            needs_layout_passes=False, use_tc_tiling_on_sc=False
        ),
        scratch_types=[
            pltpu.VMEM((rows, n_dim), jnp.float32),
            pltpu.VMEM((xnt.size,), jnp.float32),
            pltpu.VMEM((fpt.size,), jnp.float32),
            pltpu.VMEM((rows, n_dim), jnp.float32),
            pltpu.VMEM((rows, n_dim), jnp.float32),
            pltpu.SemaphoreType.DMA,
        ],
    )(x128, xnt, fpt)
    return f128[:, :n_dim], t128[:, :n_dim]
